# baseline (speedup 1.0000x reference)
import functools

import jax
import jax.numpy as jnp
import numpy as np
from jax import lax
from jax.experimental import pallas as pl
from jax.experimental.pallas import tpu as pltpu
from jax.experimental.pallas import tpu_sc as plsc

T, DEPTH, HEIGHT, WIDTH = 3, 72, 512, 512
N = 1048576

NC, NS, L = 2, 16, 16
NW = NC * NS
CHUNK = 128
PER_TILE = N // NW
NCHUNKS = PER_TILE // CHUNK
VPC = CHUNK // L

PY = WIDTH // 2
PZ = HEIGHT * WIDTH // 2
NROWS = DEPTH * PZ
MP = DEPTH * HEIGHT * WIDTH


def _lerp(a, b, w):
  return a + w * (b - a)


def _body(crs, fr, out, cbuf0, cbuf1, wbuf, ibuf, tbuf, vbuf, obuf,
          sem0, sem1, csem0, csem1):
  sems = (sem0, sem1)
  csems = (csem0, csem1)
  cbufs = (cbuf0, cbuf1)
  wid = lax.axis_index("s") * NC + lax.axis_index("c")
  chunk0 = wid * NCHUNKS
  lane = lax.iota(jnp.int32, L)

  def start_coords(j, b):
    jj = chunk0 + jnp.minimum(j, NCHUNKS - 1)
    src = crs.at[:, pl.ds(jj * CHUNK, CHUNK)]
    pltpu.async_copy(src, cbufs[b], csems[b])

  def wait_coords(b):
    pltpu.make_async_copy(crs.at[:, pl.ds(0, CHUNK)], cbufs[b], csems[b]).wait()

  def fire(j, b):
    wait_coords(b)
    start_coords(j + 1, 1 - b)
    cb = cbufs[b]
    for v in range(VPC):
      s = pl.ds(v * L, L)
      z = cb[0, s]
      y = cb[1, s]
      x = cb[2, s]
      t = cb[3, s]
      sz = z * float(DEPTH - 1)
      sy = y * float(HEIGHT - 1)
      sx = x * float(WIDTH - 1)
      ta = t * float(T)
      iz = sz.astype(jnp.int32)
      iy = sy.astype(jnp.int32)
      ix = sx.astype(jnp.int32)
      it = ta.astype(jnp.int32)
      wbuf[b, 0, s] = sz - iz.astype(jnp.float32)
      wbuf[b, 1, s] = sy - iy.astype(jnp.float32)
      wbuf[b, 2, s] = sx - ix.astype(jnp.float32)
      z0 = jnp.clip(iz, 0, DEPTH - 1)
      y0 = jnp.clip(iy, 0, HEIGHT - 1)
      x0 = jnp.clip(ix, 0, WIDTH - 1)
      t0 = jnp.clip(it, 0, T - 1)
      wbuf[b, 3, s] = ta - t0.astype(jnp.float32)
      z1 = jnp.minimum(z0 + 1, DEPTH - 1)
      y1 = jnp.minimum(y0 + 1, HEIGHT - 1)
      t1 = jnp.minimum(t0 + 1, T - 1)
      tbuf[b, 0, s] = t0
      tbuf[b, 1, s] = t1
      tbuf[b, 2, s] = (x0 & 1) * 4
      xh = lax.shift_right_logical(x0, 1)
      zb = (z0 * PZ, z1 * PZ)
      yb = (y0 * PY, y1 * PY)
      def remap(p):
        m = lax.shift_right_logical(p, 4) & 3
        return m * (MP // 8) + lax.shift_right_logical(p, 6) * 16 + (p & 15)

      for zi in range(2):
        for yi in range(2):
          pr = zb[zi] + yb[yi] + xh
          c = zi * 2 + yi
          ibuf[b, 2 * c, s] = remap(pr)
          ibuf[b, 2 * c + 1, s] = remap(jnp.minimum(pr + 1, NROWS - 1))
    for c in range(8):
      pltpu.async_copy(fr.at[ibuf.at[b, c]], vbuf.at[b, c], sems[b])

  def combine(j, b):
    for c in range(8):
      pltpu.make_async_copy(fr.at[ibuf.at[b, c]], vbuf.at[b, c], sems[b]).wait()
    for v in range(VPC):
      s = pl.ds(v * L, L)
      row = v * L + lane
      fz = wbuf[b, 0, s]
      fy = wbuf[b, 1, s]
      fx = wbuf[b, 2, s]
      ft = wbuf[b, 3, s]
      t0v = tbuf[b, 0, s]
      t1v = tbuf[b, 1, s]
      p4 = tbuf[b, 2, s]
      even = p4 == 0
      cx = []
      for c in range(4):
        va = vbuf.at[b, 2 * c]
        vb = vbuf.at[b, 2 * c + 1]
        x0_0 = plsc.load_gather(va, [row, p4 + t0v])
        x0_1 = plsc.load_gather(va, [row, p4 + t1v])
        a1_0 = plsc.load_gather(va, [row, t0v + 4])
        a1_1 = plsc.load_gather(va, [row, t1v + 4])
        b1_0 = plsc.load_gather(vb, [row, t0v])
        b1_1 = plsc.load_gather(vb, [row, t1v])
        x1_0 = jnp.where(even, a1_0, b1_0)
        x1_1 = jnp.where(even, a1_1, b1_1)
        v_x0 = _lerp(x0_0, x0_1, ft)
        v_x1 = _lerp(x1_0, x1_1, ft)
        cx.append(_lerp(v_x0, v_x1, fx))
      c0 = _lerp(cx[0], cx[1], fy)
      c1 = _lerp(cx[2], cx[3], fy)
      obuf[pl.ds(j * CHUNK + v * L, L)] = _lerp(c0, c1, fz)

  start_coords(0, 0)
  fire(0, 0)

  def loop_body(i, carry):
    j = 2 * i
    fire(j + 1, 1)
    combine(j, 0)
    fire(j + 2, 0)
    combine(j + 1, 1)
    return carry

  lax.fori_loop(0, NCHUNKS // 2 - 1, loop_body, 0)
  fire(NCHUNKS - 1, 1)
  combine(NCHUNKS - 2, 0)
  combine(NCHUNKS - 1, 1)
  wait_coords(0)
  pltpu.sync_copy(obuf, out.at[pl.ds(wid * PER_TILE, PER_TILE)])


BLK_S = 1024
IN_ROWS = MP // 128
N_BLK = IN_ROWS // BLK_S

_P = np.zeros((3, 32, 128), np.float32)
for _t in range(3):
  for _i in range(32):
    _P[_t, _i, 8 * (_i >> 1) + 4 * (_i & 1) + _t] = 1.0


def _interleave_body(f3_ref, p_ref, out_ref):
  a = f3_ref[...].astype(jnp.bfloat16)
  p = p_ref[...]
  for m in range(4):
    lo, hi = 32 * m, 32 * m + 32
    a96 = jnp.concatenate(
        [a[0, :, lo:hi], a[1, :, lo:hi], a[2, :, lo:hi]], axis=1)
    out_ref[m] = jnp.dot(a96, p, preferred_element_type=jnp.float32)


def _interleave(frames):
  f3 = frames.reshape(3, IN_ROWS, 128)
  return pl.pallas_call(
      _interleave_body,
      grid=(N_BLK,),
      in_specs=[
          pl.BlockSpec((3, BLK_S, 128), lambda k: (0, k, 0)),
          pl.BlockSpec((96, 128), lambda k: (0, 0)),
      ],
      out_specs=pl.BlockSpec((4, BLK_S, 128), lambda k: (0, k, 0)),
      out_shape=jax.ShapeDtypeStruct((4, IN_ROWS, 128), jnp.float32),
  )(f3, jnp.asarray(_P.reshape(96, 128), dtype=jnp.bfloat16))


@jax.jit
def kernel(coords, frames):
  crs = coords.T
  fr = _interleave(frames).reshape(NROWS, 8)
  mesh = plsc.VectorSubcoreMesh(core_axis_name="c", subcore_axis_name="s")
  out = pl.kernel(
      _body,
      out_type=jax.ShapeDtypeStruct((N,), jnp.float32),
      mesh=mesh,
      compiler_params=pltpu.CompilerParams(
          needs_layout_passes=False, use_tc_tiling_on_sc=False),
      scratch_types=[
          pltpu.VMEM((4, CHUNK), jnp.float32),
          pltpu.VMEM((4, CHUNK), jnp.float32),
          pltpu.VMEM((2, 4, CHUNK), jnp.float32),
          pltpu.VMEM((2, 8, CHUNK), jnp.int32),
          pltpu.VMEM((2, 3, CHUNK), jnp.int32),
          pltpu.VMEM((2, 8, CHUNK, 8), jnp.float32),
          pltpu.VMEM((PER_TILE,), jnp.float32),
          pltpu.SemaphoreType.DMA,
          pltpu.SemaphoreType.DMA,
          pltpu.SemaphoreType.DMA,
          pltpu.SemaphoreType.DMA,
      ],
  )(crs, fr)
  return out[:, None]


# speedup vs baseline: 1.1253x; 1.1253x over previous
import functools

import jax
import jax.numpy as jnp
from jax import lax
from jax.experimental import pallas as pl
from jax.experimental.pallas import tpu as pltpu
from jax.experimental.pallas import tpu_sc as plsc

T, DEPTH, HEIGHT, WIDTH = 3, 72, 512, 512
N = 1048576

NC, NS, L = 2, 16, 16
NW = NC * NS
CHUNK = 128
PER_TILE = N // NW
NCHUNKS = PER_TILE // CHUNK
VPC = CHUNK // L

DY = WIDTH
DZ = HEIGHT * WIDTH
DT = DEPTH * HEIGHT * WIDTH


def _lerp(a, b, w):
  return a + w * (b - a)


def _body(crs, fr, out, cbuf0, cbuf1, wbuf, ibuf, vbuf, obuf, sem0, sem1, csem0, csem1):
  sems = (sem0, sem1)
  csems = (csem0, csem1)
  cbufs = (cbuf0, cbuf1)
  wid = lax.axis_index("s") * NC + lax.axis_index("c")
  chunk0 = wid * NCHUNKS

  def start_coords(j, b):
    jj = chunk0 + jnp.minimum(j, NCHUNKS - 1)
    src = crs.at[:, pl.ds(jj * CHUNK, CHUNK)]
    pltpu.async_copy(src, cbufs[b], csems[b])

  def wait_coords(b):
    pltpu.make_async_copy(crs.at[:, pl.ds(0, CHUNK)], cbufs[b], csems[b]).wait()

  def fire(j, b):
    wait_coords(b)
    start_coords(j + 1, 1 - b)
    cb = cbufs[b]
    for v in range(VPC):
      s = pl.ds(v * L, L)
      z = cb[0, s]
      y = cb[1, s]
      x = cb[2, s]
      t = cb[3, s]
      sz = z * float(DEPTH - 1)
      sy = y * float(HEIGHT - 1)
      sx = x * float(WIDTH - 1)
      ta = t * float(T)
      iz = sz.astype(jnp.int32)
      iy = sy.astype(jnp.int32)
      ix = sx.astype(jnp.int32)
      it = ta.astype(jnp.int32)
      wbuf[b, 0, s] = sz - iz.astype(jnp.float32)
      wbuf[b, 1, s] = sy - iy.astype(jnp.float32)
      wbuf[b, 2, s] = sx - ix.astype(jnp.float32)
      z0 = jnp.clip(iz, 0, DEPTH - 1)
      y0 = jnp.clip(iy, 0, HEIGHT - 1)
      x0 = jnp.clip(ix, 0, WIDTH - 1)
      t0 = jnp.clip(it, 0, T - 1)
      wbuf[b, 3, s] = ta - t0.astype(jnp.float32)
      z1 = jnp.minimum(z0 + 1, DEPTH - 1)
      y1 = jnp.minimum(y0 + 1, HEIGHT - 1)
      x1 = jnp.minimum(x0 + 1, WIDTH - 1)
      t1 = jnp.minimum(t0 + 1, T - 1)
      tb = (t0 * DT, t1 * DT)
      zb = (z0 * DZ, z1 * DZ)
      yb = (y0 * DY, y1 * DY)
      xs = (x0, x1)
      for zi in range(2):
        for yi in range(2):
          sp = zb[zi] + yb[yi]
          for ti in range(2):
            base = tb[ti] + sp
            for xi in range(2):
              ibuf[b, ti * 8 + zi * 4 + yi * 2 + xi, s] = base + xs[xi]
    for c in range(16):
      pltpu.async_copy(fr.at[ibuf.at[b, c]], vbuf.at[b, c], sems[b])

  def combine(j, b):
    for c in range(16):
      pltpu.make_async_copy(fr.at[ibuf.at[b, c]], vbuf.at[b, c], sems[b]).wait()
    for v in range(VPC):
      s = pl.ds(v * L, L)
      fz = wbuf[b, 0, s]
      fy = wbuf[b, 1, s]
      fx = wbuf[b, 2, s]
      ft = wbuf[b, 3, s]
      vals = []
      for ti in range(2):
        c00 = _lerp(vbuf[b, ti * 8 + 0, s], vbuf[b, ti * 8 + 1, s], fx)
        c01 = _lerp(vbuf[b, ti * 8 + 2, s], vbuf[b, ti * 8 + 3, s], fx)
        c10 = _lerp(vbuf[b, ti * 8 + 4, s], vbuf[b, ti * 8 + 5, s], fx)
        c11 = _lerp(vbuf[b, ti * 8 + 6, s], vbuf[b, ti * 8 + 7, s], fx)
        c0 = _lerp(c00, c01, fy)
        c1 = _lerp(c10, c11, fy)
        vals.append(_lerp(c0, c1, fz))
      obuf[pl.ds(j * CHUNK + v * L, L)] = _lerp(vals[0], vals[1], ft)

  start_coords(0, 0)
  fire(0, 0)

  def loop_body(i, carry):
    j = 2 * i
    fire(j + 1, 1)
    combine(j, 0)
    fire(j + 2, 0)
    combine(j + 1, 1)
    return carry

  lax.fori_loop(0, NCHUNKS // 2 - 1, loop_body, 0)
  fire(NCHUNKS - 1, 1)
  combine(NCHUNKS - 2, 0)
  combine(NCHUNKS - 1, 1)
  wait_coords(0)
  pltpu.sync_copy(obuf, out.at[pl.ds(wid * PER_TILE, PER_TILE)])


@jax.jit
def kernel(coords, frames):
  crs = coords.T
  fr = frames.reshape(-1)
  mesh = plsc.VectorSubcoreMesh(core_axis_name="c", subcore_axis_name="s")
  out = pl.kernel(
      _body,
      out_type=jax.ShapeDtypeStruct((N,), jnp.float32),
      mesh=mesh,
      compiler_params=pltpu.CompilerParams(needs_layout_passes=False),
      scratch_types=[
          pltpu.VMEM((4, CHUNK), jnp.float32),
          pltpu.VMEM((4, CHUNK), jnp.float32),
          pltpu.VMEM((2, 4, CHUNK), jnp.float32),
          pltpu.VMEM((2, 16, CHUNK), jnp.int32),
          pltpu.VMEM((2, 16, CHUNK), jnp.float32),
          pltpu.VMEM((PER_TILE,), jnp.float32),
          pltpu.SemaphoreType.DMA,
          pltpu.SemaphoreType.DMA,
          pltpu.SemaphoreType.DMA,
          pltpu.SemaphoreType.DMA,
      ],
  )(crs, fr)
  return out[:, None]
